# baseline (speedup 1.0000x reference)
import functools

import jax
import jax.numpy as jnp
from jax import lax
from jax.experimental import pallas as pl
from jax.experimental.pallas import tpu as pltpu
from jax.experimental.pallas import tpu_sc as plsc

N_BINS = 150
BIN_PAD = 152
NC, NS, L = 2, 16, 16
NW = NC * NS
NH = 8
HROW = L * BIN_PAD
ROWS_PER_CHUNK = 32
MAGIC = float(1.5 * 2**23)
MAGIC_BITS = 0x4B400000


def _sc_partial_hists(target):
    _, n_rows, n_cols = target.shape
    n_chunks = n_rows // ROWS_PER_CHUNK
    assert n_rows % ROWS_PER_CHUNK == 0 and n_cols % (NH * L) == 0

    mesh = plsc.VectorSubcoreMesh(core_axis_name="c", subcore_axis_name="s")

    @functools.partial(
        pl.kernel,
        out_type=jax.ShapeDtypeStruct((NW, NH, HROW), jnp.float32),
        mesh=mesh,
        compiler_params=pltpu.CompilerParams(needs_layout_passes=False),
        scratch_types=[
            pltpu.VMEM((ROWS_PER_CHUNK, n_cols), jnp.float32),
            pltpu.VMEM((ROWS_PER_CHUNK, n_cols), jnp.float32),
            pltpu.VMEM((NH, HROW), jnp.float32),
            pltpu.SemaphoreType.DMA,
            pltpu.SemaphoreType.DMA,
        ],
    )
    def sc_hist(x_hbm, out_hbm, buf0, buf1, hist, sem0, sem1):
        wid = lax.axis_index("s") * NC + lax.axis_index("c")
        bufs = (buf0, buf1)
        sems = (sem0, sem1)

        zeros = jnp.zeros((L,), jnp.float32)

        for h in range(NH):
            def zero_body(i, carry, h=h):
                hist[h, pl.ds(i * L, L)] = zeros
                return carry

            lax.fori_loop(0, HROW // L, zero_body, 0)

        lane_off = lax.iota(jnp.int32, L) * BIN_PAD - MAGIC_BITS
        rows = [jnp.full((L,), h, jnp.int32) for h in range(NH)]
        ones = jnp.ones((L,), jnp.float32)
        magic = jnp.full((L,), MAGIC, jnp.float32)

        def copy(c):
            b = c % 2
            return pltpu.make_async_copy(
                x_hbm.at[wid, pl.ds(c * ROWS_PER_CHUNK, ROWS_PER_CHUNK)],
                bufs[b],
                sems[b],
            )

        def process(buf):
            groups_per_row = n_cols // (NH * L)

            def body(j, carry):
                r = j // groups_per_row
                c0 = (j % groups_per_row) * (NH * L)
                for k in range(NH):
                    v = buf[r, pl.ds(c0 + k * L, L)]
                    idx = plsc.bitcast(v + magic, jnp.int32) + lane_off
                    plsc.addupdate_scatter(hist, [rows[k], idx], ones)
                return carry

            lax.fori_loop(0, ROWS_PER_CHUNK * groups_per_row, body, 0)

        copy(0).start()
        for c in range(n_chunks):
            if c + 1 < n_chunks:
                copy(c + 1).start()
            copy(c).wait()
            process(bufs[c % 2])

        pltpu.sync_copy(hist, out_hbm.at[wid])

    return sc_hist(target)


def _tc_finish(partials):
    def body(p_ref, o_ref):
        h = jnp.sum(p_ref[...], axis=0)
        total = jnp.sum(h)
        norm = h / total + 1.1
        w = 1.0 / jnp.log(norm)
        w = jnp.where(h == 0.0, 0.0, w)
        o_ref[...] = w.reshape(1, BIN_PAD)

    return pl.pallas_call(
        body,
        out_shape=jax.ShapeDtypeStruct((1, BIN_PAD), jnp.float32),
    )(partials)


def kernel(target, n_classes):
    partials = _sc_partial_hists(target)
    weights = _tc_finish(partials.reshape(NW * NH * L, BIN_PAD))
    return weights[0, :N_BINS]


# speedup vs baseline: 1.2897x; 1.2897x over previous
import functools

import jax
import jax.numpy as jnp
from jax import lax
from jax.experimental import pallas as pl
from jax.experimental.pallas import tpu as pltpu
from jax.experimental.pallas import tpu_sc as plsc

N_BINS = 150
NC, NS, L = 2, 16, 16
NW = NC * NS
NH = 8
HROW = 2432
ROWS_PER_CHUNK = 32
MAGIC = float(1.5 * 2**23)
SHIFT_BIAS = 1275068416


def _sc_partial_hists(target):
    _, n_rows, n_cols = target.shape
    n_chunks = n_rows // ROWS_PER_CHUNK
    assert n_rows % ROWS_PER_CHUNK == 0 and n_cols % (NH * L) == 0

    mesh = plsc.VectorSubcoreMesh(core_axis_name="c", subcore_axis_name="s")

    @functools.partial(
        pl.kernel,
        out_type=jax.ShapeDtypeStruct((NW, NH, HROW), jnp.float32),
        mesh=mesh,
        compiler_params=pltpu.CompilerParams(needs_layout_passes=False),
        scratch_types=[
            pltpu.VMEM((ROWS_PER_CHUNK, n_cols), jnp.float32),
            pltpu.VMEM((ROWS_PER_CHUNK, n_cols), jnp.float32),
            pltpu.VMEM((NH, HROW), jnp.float32),
            [pltpu.VMEM((HROW,), jnp.float32) for _ in range(NH)],
            pltpu.SemaphoreType.DMA,
            pltpu.SemaphoreType.DMA,
        ],
    )
    def sc_hist(x_hbm, out_hbm, buf0, buf1, staging, hists, sem0, sem1):
        wid = lax.axis_index("s") * NC + lax.axis_index("c")
        bufs = (buf0, buf1)

        zeros = jnp.zeros((L,), jnp.float32)
        for h in range(NH):
            def zero_body(i, carry, h=h):
                hists[h][pl.ds(i * L, L)] = zeros
                return carry

            lax.fori_loop(0, HROW // L, zero_body, 0)

        col_bias = lax.iota(jnp.int32, L) + SHIFT_BIAS
        ones = jnp.ones((L,), jnp.float32)
        magic = jnp.full((L,), MAGIC, jnp.float32)

        def copy(c):
            b = c % 2
            return pltpu.make_async_copy(
                x_hbm.at[wid, pl.ds(c * ROWS_PER_CHUNK, ROWS_PER_CHUNK)],
                bufs[b],
                (sem0, sem1)[b],
            )

        def process(buf):
            groups_per_row = n_cols // (NH * L)

            def body(j, carry):
                r = j // groups_per_row
                c0 = (j % groups_per_row) * (NH * L)
                for k in range(NH):
                    v = buf[r, pl.ds(c0 + k * L, L)]
                    idx = lax.shift_left(
                        plsc.bitcast(v + magic, jnp.int32), 4
                    ) + col_bias
                    plsc.addupdate_scatter(hists[k], [idx], ones)
                return carry

            lax.fori_loop(0, ROWS_PER_CHUNK * groups_per_row, body, 0)

        copy(0).start()
        for c in range(n_chunks):
            if c + 1 < n_chunks:
                copy(c + 1).start()
            copy(c).wait()
            process(bufs[c % 2])

        for k in range(NH):
            def stage_body(i, carry, k=k):
                staging[k, pl.ds(i * L, L)] = hists[k][pl.ds(i * L, L)]
                return carry

            lax.fori_loop(0, HROW // L, stage_body, 0)

        pltpu.sync_copy(staging, out_hbm.at[wid])

    return sc_hist(target)


def _tc_finish(partials):
    def body(p_ref, o_ref):
        s = jnp.sum(p_ref[...], axis=0)
        m = s.reshape(HROW // 128, 128)
        li = lax.broadcasted_iota(jnp.int32, (128, 8), 0)
        ci = lax.broadcasted_iota(jnp.int32, (128, 8), 1)
        fold = jnp.where(li // L == ci, 1.0, 0.0)
        h = jnp.dot(m, fold, preferred_element_type=jnp.float32)
        total = jnp.sum(h)
        norm = h / total + 1.1
        w = 1.0 / jnp.log(norm)
        o_ref[...] = jnp.where(h == 0.0, 0.0, w)

    return pl.pallas_call(
        body,
        out_shape=jax.ShapeDtypeStruct((HROW // 128, 8), jnp.float32),
    )(partials)


def kernel(target, n_classes):
    partials = _sc_partial_hists(target)
    weights = _tc_finish(partials.reshape(NW * NH, HROW))
    return weights.reshape(-1)[:N_BINS]


# speedup vs baseline: 5.2424x; 4.0649x over previous
import functools

import jax
import jax.numpy as jnp
from jax import lax
from jax.experimental import pallas as pl
from jax.experimental.pallas import tpu as pltpu
from jax.experimental.pallas import tpu_sc as plsc

N_BINS = 150
NC, NS, L = 2, 16, 16
NW = NC * NS
NH = 8
HROW = 2432
ROWS_PER_CHUNK = 32
MAGIC = float(1.5 * 2**23)
SHIFT_BIAS = 1275068416


def _sc_partial_hists(target):
    _, n_rows, n_cols = target.shape
    n_chunks = n_rows // ROWS_PER_CHUNK
    assert n_rows % ROWS_PER_CHUNK == 0 and n_cols % (NH * L) == 0

    mesh = plsc.VectorSubcoreMesh(core_axis_name="c", subcore_axis_name="s")

    @functools.partial(
        pl.kernel,
        out_type=jax.ShapeDtypeStruct((NW, NH, HROW), jnp.float32),
        mesh=mesh,
        compiler_params=pltpu.CompilerParams(needs_layout_passes=False),
        scratch_types=[
            pltpu.VMEM((ROWS_PER_CHUNK, n_cols), jnp.float32),
            pltpu.VMEM((ROWS_PER_CHUNK, n_cols), jnp.float32),
            pltpu.VMEM((NH, HROW), jnp.float32),
            [pltpu.VMEM((HROW,), jnp.float32) for _ in range(NH)],
            pltpu.SemaphoreType.DMA,
            pltpu.SemaphoreType.DMA,
        ],
    )
    def sc_hist(x_hbm, out_hbm, buf0, buf1, staging, hists, sem0, sem1):
        wid = lax.axis_index("s") * NC + lax.axis_index("c")
        bufs = (buf0, buf1)

        zeros = jnp.zeros((L,), jnp.float32)
        for h in range(NH):
            def zero_body(i, carry, h=h):
                hists[h][pl.ds(i * L, L)] = zeros
                return carry

            lax.fori_loop(0, HROW // L, zero_body, 0)

        col_bias = lax.iota(jnp.int32, L) + SHIFT_BIAS
        ones = jnp.ones((L,), jnp.float32)
        magic = jnp.full((L,), MAGIC, jnp.float32)

        def copy(c):
            b = c % 2
            return pltpu.make_async_copy(
                x_hbm.at[wid, pl.ds(c * ROWS_PER_CHUNK, ROWS_PER_CHUNK)],
                bufs[b],
                (sem0, sem1)[b],
            )

        def process(buf):
            groups_per_row = n_cols // (NH * L)

            def body(j, carry):
                r = j // groups_per_row
                c0 = (j % groups_per_row) * (NH * L)
                vs = [buf[r, pl.ds(c0 + k * L, L)] for k in range(NH)]
                idxs = [
                    lax.shift_left(plsc.bitcast(v + magic, jnp.int32), 4)
                    + col_bias
                    for v in vs
                ]
                for k in range(NH):
                    plsc.addupdate_scatter(hists[k], [idxs[k]], ones)
                return carry

            lax.fori_loop(0, ROWS_PER_CHUNK * groups_per_row, body, 0)

        copy(0).start()
        for c in range(n_chunks):
            if c + 1 < n_chunks:
                copy(c + 1).start()
            copy(c).wait()
            process(bufs[c % 2])

        for k in range(NH):
            def stage_body(i, carry, k=k):
                staging[k, pl.ds(i * L, L)] = hists[k][pl.ds(i * L, L)]
                return carry

            lax.fori_loop(0, HROW // L, stage_body, 0)

        pltpu.sync_copy(staging, out_hbm.at[wid])

    return sc_hist(target)


def _tc_finish(partials):
    def body(p_ref, o_ref):
        s = jnp.sum(p_ref[...], axis=0)
        m = s.reshape(HROW // 128, 128)
        li = lax.broadcasted_iota(jnp.int32, (128, 8), 0)
        ci = lax.broadcasted_iota(jnp.int32, (128, 8), 1)
        fold = jnp.where(li // L == ci, 1.0, 0.0)
        h = jnp.dot(m, fold, preferred_element_type=jnp.float32)
        total = jnp.sum(h)
        norm = h / total + 1.1
        w = 1.0 / jnp.log(norm)
        o_ref[...] = jnp.where(h == 0.0, 0.0, w)

    return pl.pallas_call(
        body,
        out_shape=jax.ShapeDtypeStruct((HROW // 128, 8), jnp.float32),
    )(partials)


def kernel(target, n_classes):
    partials = _sc_partial_hists(target)
    weights = _tc_finish(partials.reshape(NW * NH, HROW))
    return weights.reshape(-1)[:N_BINS]
